# Initial kernel scaffold; baseline (speedup 1.0000x reference)
import functools

import jax
import jax.numpy as jnp
from jax import lax
from jax.experimental import pallas as pl
from jax.experimental.pallas import tpu as pltpu
from jax.experimental.pallas import tpu_sc as plsc

F32 = jnp.float32
WIDE = 144
CHUNK = 80


def _pre_body(h_ref, x_ref, w1a_ref, w1b_ref, b1_ref, t1_ref, t2_ref):
    h = h_ref[...]
    x = x_ref[...]
    bn = h.shape[0]
    pad = jnp.zeros((bn, WIDE - 128 - 3), dtype=F32)
    t1 = jnp.dot(h, w1a_ref[...], preferred_element_type=F32) + b1_ref[...]
    t2 = jnp.dot(h, w1b_ref[...], preferred_element_type=F32)
    t1_ref[...] = jnp.concatenate([t1, x, pad], axis=1)
    t2_ref[...] = jnp.concatenate([t2, -x, pad], axis=1)


def _k_pre(h, x, w1a, w1b, b1, n):
    bn = 2000
    grid = (n // bn,)
    return pl.pallas_call(
        _pre_body,
        grid=grid,
        in_specs=[
            pl.BlockSpec((bn, 128), lambda i: (i, 0)),
            pl.BlockSpec((bn, 3), lambda i: (i, 0)),
            pl.BlockSpec((128, 128), lambda i: (0, 0)),
            pl.BlockSpec((128, 128), lambda i: (0, 0)),
            pl.BlockSpec((128,), lambda i: (0,)),
        ],
        out_specs=[
            pl.BlockSpec((bn, WIDE), lambda i: (i, 0)),
            pl.BlockSpec((bn, WIDE), lambda i: (i, 0)),
        ],
        out_shape=[
            jax.ShapeDtypeStruct((n, WIDE), F32),
            jax.ShapeDtypeStruct((n, WIDE), F32),
        ],
    )(h, x, w1a, w1b, b1)


def _make_gather(e, n, nc, ns):
    nw = nc * ns
    epw = e // nw
    nchunks = epw // CHUNK
    mesh = plsc.VectorSubcoreMesh(core_axis_name="c", subcore_axis_name="s")

    @functools.partial(
        pl.kernel,
        mesh=mesh,
        out_type=[
            jax.ShapeDtypeStruct((e, WIDE), F32),
            jax.ShapeDtypeStruct((e, WIDE), F32),
        ],
        scratch_types=[
            pltpu.VMEM((CHUNK,), jnp.int32),
            pltpu.VMEM((CHUNK,), jnp.int32),
            pltpu.VMEM((CHUNK, WIDE), F32),
            pltpu.VMEM((CHUNK, WIDE), F32),
            pltpu.SemaphoreType.DMA,
            pltpu.SemaphoreType.DMA,
        ],
    )
    def gather_k(t1_hbm, t2_hbm, ii_hbm, jj_hbm, g1_hbm, g2_hbm,
                 ii_v, jj_v, g1_v, g2_v, sem1, sem2):
        cid = lax.axis_index("c")
        sid = lax.axis_index("s")
        wid = sid * nc + cid
        wbase = wid * epw

        def chunk(k, carry):
            base = wbase + k * CHUNK
            pltpu.sync_copy(ii_hbm.at[pl.ds(base, CHUNK)], ii_v)
            pltpu.sync_copy(jj_hbm.at[pl.ds(base, CHUNK)], jj_v)
            d1 = pltpu.async_copy(t1_hbm.at[ii_v], g1_v, sem1)
            d2 = pltpu.async_copy(t2_hbm.at[jj_v], g2_v, sem2)
            d1.wait()
            d2.wait()
            pltpu.sync_copy(g1_v, g1_hbm.at[pl.ds(base, CHUNK)])
            pltpu.sync_copy(g2_v, g2_hbm.at[pl.ds(base, CHUNK)])
            return carry

        lax.fori_loop(0, nchunks, chunk, 0)

    return gather_k


def _edge_body(g1_ref, g2_ref, w1r_ref, w2_ref, b2_ref, m_ref):
    g = g1_ref[...] + g2_ref[...]
    bn = g.shape[0]
    t = g[:, :128]
    rel = g[:, 128:131]
    dist = jnp.sqrt(jnp.sum(rel * rel, axis=1, keepdims=True))
    centers = jnp.linspace(0.0, 5.0, 32, dtype=F32)
    rbf = jnp.exp(-10.0 * (dist - centers) ** 2)
    t = t + jnp.dot(rbf, w1r_ref[...], preferred_element_type=F32)
    m = jnp.dot(jax.nn.relu(t), w2_ref[...], preferred_element_type=F32)
    m = m + b2_ref[...]
    ones = jnp.ones((bn, 1), dtype=F32)
    pad = jnp.zeros((bn, WIDE - 129), dtype=F32)
    m_ref[...] = jnp.concatenate([m, ones, pad], axis=1)


def _k_edge(g1, g2, w1r, w2, b2, e):
    bn = 2000
    grid = (e // bn,)
    return pl.pallas_call(
        _edge_body,
        grid=grid,
        in_specs=[
            pl.BlockSpec((bn, WIDE), lambda i: (i, 0)),
            pl.BlockSpec((bn, WIDE), lambda i: (i, 0)),
            pl.BlockSpec((32, 128), lambda i: (0, 0)),
            pl.BlockSpec((128, 128), lambda i: (0, 0)),
            pl.BlockSpec((128,), lambda i: (0,)),
        ],
        out_specs=pl.BlockSpec((bn, WIDE), lambda i: (i, 0)),
        out_shape=jax.ShapeDtypeStruct((e, WIDE), F32),
    )(g1, g2, w1r, w2, b2)


def _make_scatter(e, n, nc, ns):
    nw = nc * ns
    epw = e // nw
    nchunks = epw // CHUNK
    rows_per_sub = n // ns
    zrows = 125
    mesh = plsc.VectorSubcoreMesh(core_axis_name="c", subcore_axis_name="s")

    @functools.partial(
        pl.kernel,
        mesh=mesh,
        out_type=jax.ShapeDtypeStruct((nc, n, WIDE), F32),
        scratch_types=[
            pltpu.VMEM((CHUNK,), jnp.int32),
            pltpu.VMEM((CHUNK, WIDE), F32),
            pltpu.VMEM((zrows, WIDE), F32),
            pltpu.VMEM_SHARED((n, WIDE), F32),
        ],
    )
    def scatter_k(m_hbm, ii_hbm, out_hbm, ii_v, m_v, z_v, acc_sh):
        cid = lax.axis_index("c")
        sid = lax.axis_index("s")
        wid = sid * nc + cid
        wbase = wid * epw

        zvec = jnp.zeros((16,), dtype=F32)

        def zrow(r, carry):
            for c in range(WIDE // 16):
                z_v[r, pl.ds(c * 16, 16)] = zvec
            return carry

        lax.fori_loop(0, zrows, zrow, 0)

        def zcopy(k, carry):
            pltpu.sync_copy(
                z_v, acc_sh.at[pl.ds(sid * rows_per_sub + k * zrows, zrows)])
            return carry

        lax.fori_loop(0, rows_per_sub // zrows, zcopy, 0)
        plsc.subcore_barrier()

        def chunk(k, carry):
            base = wbase + k * CHUNK
            pltpu.sync_copy(ii_hbm.at[pl.ds(base, CHUNK)], ii_v)
            pltpu.sync_copy(m_hbm.at[pl.ds(base, CHUNK)], m_v)
            pltpu.sync_copy(m_v, acc_sh.at[ii_v], add=True)
            return carry

        lax.fori_loop(0, nchunks, chunk, 0)
        plsc.subcore_barrier()

        def wcopy(k, carry):
            r0 = sid * rows_per_sub + k * zrows
            pltpu.sync_copy(acc_sh.at[pl.ds(r0, zrows)], z_v)
            pltpu.sync_copy(z_v, out_hbm.at[cid].at[pl.ds(r0, zrows)])
            return carry

        lax.fori_loop(0, rows_per_sub // zrows, wcopy, 0)

    return scatter_k


def _update_body(h_ref, p_ref, w3a_ref, w3b_ref, b3_ref, w4_ref, b4_ref,
                 out_ref):
    a = p_ref[0] + p_ref[1]
    aggr = a[:, :128]
    deg = a[:, 128:129]
    aggr = aggr / jnp.maximum(deg, 1.0)
    t = (jnp.dot(h_ref[...], w3a_ref[...], preferred_element_type=F32)
         + jnp.dot(aggr, w3b_ref[...], preferred_element_type=F32)
         + b3_ref[...])
    out_ref[...] = (jnp.dot(jax.nn.relu(t), w4_ref[...],
                            preferred_element_type=F32) + b4_ref[...])


def _k_update(h, p, w3a, w3b, b3, w4, b4, n):
    bn = 1000
    grid = (n // bn,)
    return pl.pallas_call(
        _update_body,
        grid=grid,
        in_specs=[
            pl.BlockSpec((bn, 128), lambda i: (i, 0)),
            pl.BlockSpec((2, bn, WIDE), lambda i: (0, i, 0)),
            pl.BlockSpec((128, 128), lambda i: (0, 0)),
            pl.BlockSpec((128, 128), lambda i: (0, 0)),
            pl.BlockSpec((128,), lambda i: (0,)),
            pl.BlockSpec((128, 128), lambda i: (0, 0)),
            pl.BlockSpec((128,), lambda i: (0,)),
        ],
        out_specs=pl.BlockSpec((bn, 128), lambda i: (i, 0)),
        out_shape=jax.ShapeDtypeStruct((n, 128), F32),
    )(h, p, w3a, w3b, b3, w4, b4)


def kernel(h, x, edge_index, W1, b1, W2, b2, W3, b3, W4, b4):
    n = h.shape[0]
    e = edge_index.shape[1]
    try:
        info = plsc.get_sparse_core_info()
        nc, ns = info.num_cores, info.num_subcores
    except Exception:
        nc, ns = 2, 16

    ii = edge_index[0].astype(jnp.int32)
    jj = edge_index[1].astype(jnp.int32)
    w1a = W1[:128]
    w1b = W1[128:256]
    w1r = W1[256:288]
    w3a = W3[:128]
    w3b = W3[128:256]

    t1, t2 = _k_pre(h, x, w1a, w1b, b1, n)
    g1, g2 = _make_gather(e, n, nc, ns)(t1, t2, ii, jj)
    m = _k_edge(g1, g2, w1r, W2, b2, e)
    p = _make_scatter(e, n, nc, ns)(m, ii)
    h_new = _k_update(h, p, w3a, w3b, b3, W4, b4, n)
    return (h_new, x)


# speedup vs baseline: 3.6988x; 3.6988x over previous
import functools

import jax
import jax.numpy as jnp
from jax import lax
from jax.experimental import pallas as pl
from jax.experimental.pallas import tpu as pltpu
from jax.experimental.pallas import tpu_sc as plsc

F32 = jnp.float32
JOB = 128


def _pre_body(h_ref, w1a_ref, w1b_ref, b1_ref, t1_ref, t2_ref):
    h = h_ref[...]
    t1_ref[...] = jnp.dot(h, w1a_ref[...], preferred_element_type=F32) \
        + b1_ref[...]
    t2_ref[...] = jnp.dot(h, w1b_ref[...], preferred_element_type=F32)


def _k_pre(h, w1a, w1b, b1, n):
    bn = 2000
    return pl.pallas_call(
        _pre_body,
        grid=(n // bn,),
        in_specs=[
            pl.BlockSpec((bn, 128), lambda i: (i, 0)),
            pl.BlockSpec((128, 128), lambda i: (0, 0)),
            pl.BlockSpec((128, 128), lambda i: (0, 0)),
            pl.BlockSpec((128,), lambda i: (0,)),
        ],
        out_specs=[
            pl.BlockSpec((bn, 128), lambda i: (i, 0)),
            pl.BlockSpec((bn, 128), lambda i: (i, 0)),
        ],
        out_shape=[
            jax.ShapeDtypeStruct((n, 128), F32),
            jax.ShapeDtypeStruct((n, 128), F32),
        ],
    )(h, w1a, w1b, b1)


def _make_gather(e, n, nc, ns):
    nw = nc * ns
    njobs = e // JOB
    jobs_per_w = (njobs + nw - 1) // nw
    mesh = plsc.VectorSubcoreMesh(core_axis_name="c", subcore_axis_name="s")

    @functools.partial(
        pl.kernel,
        mesh=mesh,
        out_type=[
            jax.ShapeDtypeStruct((e, 128), F32),
            jax.ShapeDtypeStruct((e,), F32),
        ],
        scratch_types=[
            pltpu.VMEM((JOB,), jnp.int32),
            pltpu.VMEM((JOB,), jnp.int32),
            pltpu.VMEM((JOB, 128), F32),
            pltpu.VMEM((JOB, 128), F32),
            pltpu.VMEM((4 * n,), F32),
            pltpu.VMEM((JOB,), F32),
            pltpu.SemaphoreType.DMA,
            pltpu.SemaphoreType.DMA,
        ],
        compiler_params=pltpu.CompilerParams(needs_layout_passes=False),
    )
    def gather_k(t1_hbm, t2_hbm, ii_hbm, jj_hbm, xf_hbm, g_hbm, d2_hbm,
                 ii_v, jj_v, g1_v, g2_v, xf_v, d2_v, sem1, sem2):
        cid = lax.axis_index("c")
        sid = lax.axis_index("s")
        wid = sid * nc + cid

        pltpu.sync_copy(xf_hbm, xf_v)

        def job(k, carry):
            row = wid + k * nw

            @pl.when(row < njobs)
            def _():
                base = row * JOB
                pltpu.sync_copy(ii_hbm.at[pl.ds(base, JOB)], ii_v)
                pltpu.sync_copy(jj_hbm.at[pl.ds(base, JOB)], jj_v)
                d1 = pltpu.async_copy(t1_hbm.at[ii_v], g1_v, sem1)
                d2 = pltpu.async_copy(t2_hbm.at[jj_v], g2_v, sem2)

                for s in range(JOB // 16):
                    i16 = ii_v[pl.ds(s * 16, 16)] * 4
                    j16 = jj_v[pl.ds(s * 16, 16)] * 4
                    acc = jnp.zeros((16,), dtype=F32)
                    for c in range(3):
                        xi = plsc.load_gather(xf_v, [i16 + c])
                        xj = plsc.load_gather(xf_v, [j16 + c])
                        d = xi - xj
                        acc = acc + d * d
                    d2_v[pl.ds(s * 16, 16)] = acc

                d1.wait()
                d2.wait()

                def add_row(r, carry2):
                    for c in range(8):
                        sl = pl.ds(c * 16, 16)
                        g1_v[r, sl] = g1_v[r, sl] + g2_v[r, sl]
                    return carry2

                lax.fori_loop(0, JOB, add_row, 0)
                pltpu.sync_copy(g1_v, g_hbm.at[pl.ds(base, JOB)])
                pltpu.sync_copy(d2_v, d2_hbm.at[pl.ds(base, JOB)])

            return carry

        lax.fori_loop(0, jobs_per_w, job, 0)

    return gather_k


def _edge_body(g_ref, d2_ref, w1r_ref, w2_ref, b2_ref, m_ref):
    dist = jnp.sqrt(d2_ref[...])
    centers = (lax.broadcasted_iota(jnp.int32, (1, 32), 1).astype(F32)
               * (5.0 / 31.0))
    rbf = jnp.exp(-10.0 * (dist - centers) ** 2)
    t = g_ref[...] + jnp.dot(rbf, w1r_ref[...], preferred_element_type=F32)
    m = jnp.dot(jax.nn.relu(t), w2_ref[...], preferred_element_type=F32)
    m_ref[...] = m + b2_ref[...]


def _k_edge(g, d2, w1r, w2, b2, e):
    bn = 2560
    return pl.pallas_call(
        _edge_body,
        grid=(e // bn,),
        in_specs=[
            pl.BlockSpec((bn, 128), lambda i: (i, 0)),
            pl.BlockSpec((bn, 1), lambda i: (i, 0)),
            pl.BlockSpec((32, 128), lambda i: (0, 0)),
            pl.BlockSpec((128, 128), lambda i: (0, 0)),
            pl.BlockSpec((128,), lambda i: (0,)),
        ],
        out_specs=pl.BlockSpec((bn, 128), lambda i: (i, 0)),
        out_shape=jax.ShapeDtypeStruct((e, 128), F32),
    )(g, d2, w1r, w2, b2)


def _make_scatter(e, n, nc, ns):
    nw = nc * ns
    njobs = e // JOB
    jobs_per_sub = (njobs + ns - 1) // ns
    half = n // nc
    trash = half
    zsub = 5
    zspan = half // zsub
    zrows = 200
    mesh = plsc.VectorSubcoreMesh(core_axis_name="c", subcore_axis_name="s")

    @functools.partial(
        pl.kernel,
        mesh=mesh,
        out_type=[
            jax.ShapeDtypeStruct((n, 128), F32),
            jax.ShapeDtypeStruct((nw * n,), F32),
        ],
        scratch_types=[
            pltpu.VMEM((JOB,), jnp.int32),
            pltpu.VMEM((JOB, 128), F32),
            pltpu.VMEM((zrows, 128), F32),
            pltpu.VMEM((n,), F32),
            pltpu.VMEM_SHARED((half + 8, 128), F32),
        ],
        compiler_params=pltpu.CompilerParams(needs_layout_passes=False),
    )
    def scatter_k(m_hbm, ii_hbm, out_hbm, deg_hbm,
                  ii_v, m_v, z_v, deg_v, acc_sh):
        cid = lax.axis_index("c")
        sid = lax.axis_index("s")
        wid = sid * nc + cid
        lo = cid * half

        zvec = jnp.zeros((16,), dtype=F32)

        def zrow(r, carry):
            for c in range(8):
                z_v[r, pl.ds(c * 16, 16)] = zvec
            return carry

        lax.fori_loop(0, zrows, zrow, 0)

        def zdeg(r, carry):
            deg_v[pl.ds(r * 16, 16)] = zvec
            return carry

        lax.fori_loop(0, n // 16, zdeg, 0)

        @pl.when(sid < zsub)
        def _zero():
            def zcopy(k, carry):
                pltpu.sync_copy(
                    z_v, acc_sh.at[pl.ds(sid * zspan + k * zrows, zrows)])
                return carry

            lax.fori_loop(0, zspan // zrows, zcopy, 0)

        plsc.subcore_barrier()

        def job(k, carry):
            row = sid + k * ns

            @pl.when(row < njobs)
            def _():
                base = row * JOB
                pltpu.sync_copy(ii_hbm.at[pl.ds(base, JOB)], ii_v)
                pltpu.sync_copy(m_hbm.at[pl.ds(base, JOB)], m_v)
                for s in range(JOB // 16):
                    sl = pl.ds(s * 16, 16)
                    i16 = ii_v[sl]
                    loc = i16 - lo
                    own = (loc >= 0) & (loc < half)
                    plsc.addupdate_scatter(
                        deg_v, [i16], jnp.where(own, 1.0, 0.0).astype(F32))
                    ii_v[sl] = jnp.where(own, loc, trash)
                pltpu.sync_copy(m_v, acc_sh.at[ii_v], add=True)

            return carry

        lax.fori_loop(0, jobs_per_sub, job, 0)
        plsc.subcore_barrier()

        pltpu.sync_copy(deg_v, deg_hbm.at[pl.ds(wid * n, n)])

        @pl.when(sid < zsub)
        def _writeout():
            def wcopy(k, carry):
                r0 = sid * zspan + k * zrows
                pltpu.sync_copy(acc_sh.at[pl.ds(r0, zrows)], z_v)
                pltpu.sync_copy(z_v, out_hbm.at[pl.ds(lo + r0, zrows)])
                return carry

            lax.fori_loop(0, zspan // zrows, wcopy, 0)

    return scatter_k


def _update_body(h_ref, p_ref, r_ref, w3a_ref, w3b_ref, b3_ref, w4_ref,
                 b4_ref, out_ref):
    aggr = p_ref[...] * r_ref[...]
    t = (jnp.dot(h_ref[...], w3a_ref[...], preferred_element_type=F32)
         + jnp.dot(aggr, w3b_ref[...], preferred_element_type=F32)
         + b3_ref[...])
    out_ref[...] = (jnp.dot(jax.nn.relu(t), w4_ref[...],
                            preferred_element_type=F32) + b4_ref[...])


def _k_update(h, p, recip, w3a, w3b, b3, w4, b4, n):
    bn = 1000
    return pl.pallas_call(
        _update_body,
        grid=(n // bn,),
        in_specs=[
            pl.BlockSpec((bn, 128), lambda i: (i, 0)),
            pl.BlockSpec((bn, 128), lambda i: (i, 0)),
            pl.BlockSpec((bn, 1), lambda i: (i, 0)),
            pl.BlockSpec((128, 128), lambda i: (0, 0)),
            pl.BlockSpec((128, 128), lambda i: (0, 0)),
            pl.BlockSpec((128,), lambda i: (0,)),
            pl.BlockSpec((128, 128), lambda i: (0, 0)),
            pl.BlockSpec((128,), lambda i: (0,)),
        ],
        out_specs=pl.BlockSpec((bn, 128), lambda i: (i, 0)),
        out_shape=jax.ShapeDtypeStruct((n, 128), F32),
    )(h, p, recip, w3a, w3b, b3, w4, b4)


def kernel(h, x, edge_index, W1, b1, W2, b2, W3, b3, W4, b4):
    n = h.shape[0]
    e = edge_index.shape[1]
    try:
        info = plsc.get_sparse_core_info()
        nc, ns = info.num_cores, info.num_subcores
    except Exception:
        nc, ns = 2, 16

    ii = edge_index[0].astype(jnp.int32)
    jj = edge_index[1].astype(jnp.int32)
    xf = jnp.pad(x.astype(F32), ((0, 0), (0, 1))).reshape(-1)
    w1a = W1[:128]
    w1b = W1[128:256]
    w1r = W1[256:288]
    w3a = W3[:128]
    w3b = W3[128:256]

    t1, t2 = _k_pre(h, w1a, w1b, b1, n)
    g, d2 = _make_gather(e, n, nc, ns)(t1, t2, ii, jj, xf)
    m = _k_edge(g, d2.reshape(e, 1), w1r, W2, b2, e)
    p, deg = _make_scatter(e, n, nc, ns)(m, ii)
    recip = (1.0 / jnp.clip(deg.reshape(nc * ns, n).sum(axis=0),
                            1.0, None))[:, None]
    h_new = _k_update(h, p, recip, w3a, w3b, b3, W4, b4, n)
    return (h_new, x)


# speedup vs baseline: 4.1135x; 1.1121x over previous
import functools

import jax
import jax.numpy as jnp
from jax import lax
from jax.experimental import pallas as pl
from jax.experimental.pallas import tpu as pltpu
from jax.experimental.pallas import tpu_sc as plsc

F32 = jnp.float32
JOB = 256


def _pre_body(h_ref, w1a_ref, w1b_ref, b1_ref, t1_ref, t2_ref):
    h = h_ref[...]
    t1_ref[...] = jnp.dot(h, w1a_ref[...], preferred_element_type=F32) \
        + b1_ref[...]
    t2_ref[...] = jnp.dot(h, w1b_ref[...], preferred_element_type=F32)


def _k_pre(h, w1a, w1b, b1, n):
    bn = 2000
    return pl.pallas_call(
        _pre_body,
        grid=(n // bn,),
        in_specs=[
            pl.BlockSpec((bn, 128), lambda i: (i, 0)),
            pl.BlockSpec((128, 128), lambda i: (0, 0)),
            pl.BlockSpec((128, 128), lambda i: (0, 0)),
            pl.BlockSpec((128,), lambda i: (0,)),
        ],
        out_specs=[
            pl.BlockSpec((bn, 128), lambda i: (i, 0)),
            pl.BlockSpec((bn, 128), lambda i: (i, 0)),
        ],
        out_shape=[
            jax.ShapeDtypeStruct((n, 128), F32),
            jax.ShapeDtypeStruct((n, 128), F32),
        ],
    )(h, w1a, w1b, b1)


def _make_gather(e, n, nc, ns):
    nw = nc * ns
    njobs = e // JOB
    jobs_per_w = (njobs + nw - 1) // nw
    mesh = plsc.VectorSubcoreMesh(core_axis_name="c", subcore_axis_name="s")

    @functools.partial(
        pl.kernel,
        mesh=mesh,
        out_type=[
            jax.ShapeDtypeStruct((e, 128), F32),
            jax.ShapeDtypeStruct((e,), F32),
        ],
        scratch_types=[
            pltpu.VMEM((JOB,), jnp.int32),
            pltpu.VMEM((JOB,), jnp.int32),
            pltpu.VMEM((JOB, 128), F32),
            pltpu.VMEM((JOB, 128), F32),
            pltpu.VMEM((4 * n,), F32),
            pltpu.VMEM((JOB,), F32),
            pltpu.SemaphoreType.DMA,
            pltpu.SemaphoreType.DMA,
        ],
        compiler_params=pltpu.CompilerParams(needs_layout_passes=False),
    )
    def gather_k(t1_hbm, t2_hbm, ii_hbm, jj_hbm, xf_hbm, g_hbm, d2_hbm,
                 ii_v, jj_v, g1_v, g2_v, xf_v, d2_v, sem1, sem2):
        cid = lax.axis_index("c")
        sid = lax.axis_index("s")
        wid = sid * nc + cid

        pltpu.sync_copy(xf_hbm, xf_v)

        def job(k, carry):
            row = wid + k * nw

            @pl.when(row < njobs)
            def _():
                base = row * JOB
                pltpu.sync_copy(ii_hbm.at[pl.ds(base, JOB)], ii_v)
                pltpu.sync_copy(jj_hbm.at[pl.ds(base, JOB)], jj_v)
                d1 = pltpu.async_copy(t1_hbm.at[ii_v], g1_v, sem1)
                d2 = pltpu.async_copy(t2_hbm.at[jj_v], g2_v, sem2)

                for s in range(JOB // 16):
                    i16 = ii_v[pl.ds(s * 16, 16)] * 4
                    j16 = jj_v[pl.ds(s * 16, 16)] * 4
                    acc = jnp.zeros((16,), dtype=F32)
                    for c in range(3):
                        xi = plsc.load_gather(xf_v, [i16 + c])
                        xj = plsc.load_gather(xf_v, [j16 + c])
                        d = xi - xj
                        acc = acc + d * d
                    d2_v[pl.ds(s * 16, 16)] = acc

                d1.wait()
                d2.wait()

                def add_row(r, carry2):
                    for c in range(8):
                        sl = pl.ds(c * 16, 16)
                        g1_v[r, sl] = g1_v[r, sl] + g2_v[r, sl]
                    return carry2

                lax.fori_loop(0, JOB, add_row, 0)
                pltpu.sync_copy(g1_v, g_hbm.at[pl.ds(base, JOB)])
                pltpu.sync_copy(d2_v, d2_hbm.at[pl.ds(base, JOB)])

            return carry

        lax.fori_loop(0, jobs_per_w, job, 0)

    return gather_k


def _edge_body(g_ref, d2_ref, w1r_ref, w2_ref, b2_ref, m_ref):
    dist = jnp.sqrt(d2_ref[...])
    centers = (lax.broadcasted_iota(jnp.int32, (1, 32), 1).astype(F32)
               * (5.0 / 31.0))
    rbf = jnp.exp(-10.0 * (dist - centers) ** 2)
    t = g_ref[...] + jnp.dot(rbf, w1r_ref[...], preferred_element_type=F32)
    m = jnp.dot(jax.nn.relu(t), w2_ref[...], preferred_element_type=F32)
    m_ref[...] = m + b2_ref[...]


def _k_edge(g, d2, w1r, w2, b2, e):
    bn = 2560
    return pl.pallas_call(
        _edge_body,
        grid=(e // bn,),
        in_specs=[
            pl.BlockSpec((bn, 128), lambda i: (i, 0)),
            pl.BlockSpec((bn, 1), lambda i: (i, 0)),
            pl.BlockSpec((32, 128), lambda i: (0, 0)),
            pl.BlockSpec((128, 128), lambda i: (0, 0)),
            pl.BlockSpec((128,), lambda i: (0,)),
        ],
        out_specs=pl.BlockSpec((bn, 128), lambda i: (i, 0)),
        out_shape=jax.ShapeDtypeStruct((e, 128), F32),
    )(g, d2, w1r, w2, b2)


def _make_scatter(e, n, nc, ns):
    nw = nc * ns
    njobs = e // JOB
    jobs_per_sub = (njobs + ns - 1) // ns
    half = n // nc
    trash = half
    zsub = 5
    zspan = half // zsub
    zrows = 200
    mesh = plsc.VectorSubcoreMesh(core_axis_name="c", subcore_axis_name="s")

    @functools.partial(
        pl.kernel,
        mesh=mesh,
        out_type=[
            jax.ShapeDtypeStruct((n, 128), F32),
            jax.ShapeDtypeStruct((nw * n,), F32),
        ],
        scratch_types=[
            pltpu.VMEM((JOB,), jnp.int32),
            pltpu.VMEM((JOB, 128), F32),
            pltpu.VMEM((zrows, 128), F32),
            pltpu.VMEM((n,), F32),
            pltpu.VMEM_SHARED((half + 8, 128), F32),
        ],
        compiler_params=pltpu.CompilerParams(needs_layout_passes=False),
    )
    def scatter_k(m_hbm, ii_hbm, out_hbm, deg_hbm,
                  ii_v, m_v, z_v, deg_v, acc_sh):
        cid = lax.axis_index("c")
        sid = lax.axis_index("s")
        wid = sid * nc + cid
        lo = cid * half

        zvec = jnp.zeros((16,), dtype=F32)

        def zrow(r, carry):
            for c in range(8):
                z_v[r, pl.ds(c * 16, 16)] = zvec
            return carry

        lax.fori_loop(0, zrows, zrow, 0)

        def zdeg(r, carry):
            deg_v[pl.ds(r * 16, 16)] = zvec
            return carry

        lax.fori_loop(0, n // 16, zdeg, 0)

        @pl.when(sid < zsub)
        def _zero():
            def zcopy(k, carry):
                pltpu.sync_copy(
                    z_v, acc_sh.at[pl.ds(sid * zspan + k * zrows, zrows)])
                return carry

            lax.fori_loop(0, zspan // zrows, zcopy, 0)

        plsc.subcore_barrier()

        def job(k, carry):
            row = sid + k * ns

            @pl.when(row < njobs)
            def _():
                base = row * JOB
                pltpu.sync_copy(ii_hbm.at[pl.ds(base, JOB)], ii_v)
                pltpu.sync_copy(m_hbm.at[pl.ds(base, JOB)], m_v)
                for s in range(JOB // 16):
                    sl = pl.ds(s * 16, 16)
                    i16 = ii_v[sl]
                    loc = i16 - lo
                    own = (loc >= 0) & (loc < half)
                    plsc.addupdate_scatter(
                        deg_v, [i16], jnp.where(own, 1.0, 0.0).astype(F32))
                    ii_v[sl] = jnp.where(own, loc, trash)
                pltpu.sync_copy(m_v, acc_sh.at[ii_v], add=True)

            return carry

        lax.fori_loop(0, jobs_per_sub, job, 0)
        plsc.subcore_barrier()

        pltpu.sync_copy(deg_v, deg_hbm.at[pl.ds(wid * n, n)])

        @pl.when(sid < zsub)
        def _writeout():
            def wcopy(k, carry):
                r0 = sid * zspan + k * zrows
                pltpu.sync_copy(acc_sh.at[pl.ds(r0, zrows)], z_v)
                pltpu.sync_copy(z_v, out_hbm.at[pl.ds(lo + r0, zrows)])
                return carry

            lax.fori_loop(0, zspan // zrows, wcopy, 0)

    return scatter_k


def _update_body(h_ref, p_ref, r_ref, w3a_ref, w3b_ref, b3_ref, w4_ref,
                 b4_ref, out_ref):
    aggr = p_ref[...] * r_ref[...]
    t = (jnp.dot(h_ref[...], w3a_ref[...], preferred_element_type=F32)
         + jnp.dot(aggr, w3b_ref[...], preferred_element_type=F32)
         + b3_ref[...])
    out_ref[...] = (jnp.dot(jax.nn.relu(t), w4_ref[...],
                            preferred_element_type=F32) + b4_ref[...])


def _k_update(h, p, recip, w3a, w3b, b3, w4, b4, n):
    bn = 1000
    return pl.pallas_call(
        _update_body,
        grid=(n // bn,),
        in_specs=[
            pl.BlockSpec((bn, 128), lambda i: (i, 0)),
            pl.BlockSpec((bn, 128), lambda i: (i, 0)),
            pl.BlockSpec((bn, 1), lambda i: (i, 0)),
            pl.BlockSpec((128, 128), lambda i: (0, 0)),
            pl.BlockSpec((128, 128), lambda i: (0, 0)),
            pl.BlockSpec((128,), lambda i: (0,)),
            pl.BlockSpec((128, 128), lambda i: (0, 0)),
            pl.BlockSpec((128,), lambda i: (0,)),
        ],
        out_specs=pl.BlockSpec((bn, 128), lambda i: (i, 0)),
        out_shape=jax.ShapeDtypeStruct((n, 128), F32),
    )(h, p, recip, w3a, w3b, b3, w4, b4)


def kernel(h, x, edge_index, W1, b1, W2, b2, W3, b3, W4, b4):
    n = h.shape[0]
    e = edge_index.shape[1]
    try:
        info = plsc.get_sparse_core_info()
        nc, ns = info.num_cores, info.num_subcores
    except Exception:
        nc, ns = 2, 16

    ii = edge_index[0].astype(jnp.int32)
    jj = edge_index[1].astype(jnp.int32)
    xf = jnp.pad(x.astype(F32), ((0, 0), (0, 1))).reshape(-1)
    w1a = W1[:128]
    w1b = W1[128:256]
    w1r = W1[256:288]
    w3a = W3[:128]
    w3b = W3[128:256]

    t1, t2 = _k_pre(h, w1a, w1b, b1, n)
    g, d2 = _make_gather(e, n, nc, ns)(t1, t2, ii, jj, xf)
    m = _k_edge(g, d2.reshape(e, 1), w1r, W2, b2, e)
    p, deg = _make_scatter(e, n, nc, ns)(m, ii)
    recip = (1.0 / jnp.clip(deg.reshape(nc * ns, n).sum(axis=0),
                            1.0, None))[:, None]
    h_new = _k_update(h, p, recip, w3a, w3b, b3, W4, b4, n)
    return (h_new, x)


# speedup vs baseline: 4.5255x; 1.1002x over previous
import functools

import jax
import jax.numpy as jnp
from jax import lax
from jax.experimental import pallas as pl
from jax.experimental.pallas import tpu as pltpu
from jax.experimental.pallas import tpu_sc as plsc

F32 = jnp.float32
JOB = 200


def _pre_body(h_ref, w1a_ref, w1b_ref, b1_ref, t1_ref, t2_ref):
    h = h_ref[...]
    t1_ref[...] = jnp.dot(h, w1a_ref[...], preferred_element_type=F32) \
        + b1_ref[...]
    t2_ref[...] = jnp.dot(h, w1b_ref[...], preferred_element_type=F32)


def _k_pre(h, w1a, w1b, b1, n):
    bn = 2000
    return pl.pallas_call(
        _pre_body,
        grid=(n // bn,),
        in_specs=[
            pl.BlockSpec((bn, 128), lambda i: (i, 0)),
            pl.BlockSpec((128, 128), lambda i: (0, 0)),
            pl.BlockSpec((128, 128), lambda i: (0, 0)),
            pl.BlockSpec((128,), lambda i: (0,)),
        ],
        out_specs=[
            pl.BlockSpec((bn, 128), lambda i: (i, 0)),
            pl.BlockSpec((bn, 128), lambda i: (i, 0)),
        ],
        out_shape=[
            jax.ShapeDtypeStruct((n, 128), F32),
            jax.ShapeDtypeStruct((n, 128), F32),
        ],
    )(h, w1a, w1b, b1)


def _make_d2(e, n, nc, ns):
    nw = nc * ns
    job2 = 2000
    jobs_per_w = e // job2 // nw
    mesh = plsc.VectorSubcoreMesh(core_axis_name="c", subcore_axis_name="s")

    @functools.partial(
        pl.kernel,
        mesh=mesh,
        out_type=jax.ShapeDtypeStruct((e,), F32),
        scratch_types=[
            pltpu.VMEM((job2,), jnp.int32),
            pltpu.VMEM((job2,), jnp.int32),
            pltpu.VMEM((job2,), F32),
            pltpu.VMEM((4 * n,), F32),
        ],
        compiler_params=pltpu.CompilerParams(needs_layout_passes=False),
    )
    def d2_k(ii_hbm, jj_hbm, xf_hbm, d2_hbm, ii_v, jj_v, d2_v, xf_v):
        cid = lax.axis_index("c")
        sid = lax.axis_index("s")
        wid = sid * nc + cid
        pltpu.sync_copy(xf_hbm, xf_v)

        def job(k, carry):
            base = (wid * jobs_per_w + k) * job2
            pltpu.sync_copy(ii_hbm.at[pl.ds(base, job2)], ii_v)
            pltpu.sync_copy(jj_hbm.at[pl.ds(base, job2)], jj_v)

            def step(s, carry2):
                sl = pl.ds(s * 16, 16)
                i16 = ii_v[sl] * 4
                j16 = jj_v[sl] * 4
                acc = jnp.zeros((16,), dtype=F32)
                for c in range(3):
                    xi = plsc.load_gather(xf_v, [i16 + c])
                    xj = plsc.load_gather(xf_v, [j16 + c])
                    d = xi - xj
                    acc = acc + d * d
                d2_v[sl] = acc
                return carry2

            lax.fori_loop(0, job2 // 16, step, 0)
            pltpu.sync_copy(d2_v, d2_hbm.at[pl.ds(base, job2)])
            return carry

        lax.fori_loop(0, jobs_per_w, job, 0)

    return d2_k


def _make_gather(e, n, nc, ns):
    nw = nc * ns
    njobs = e // JOB
    jobs_per_w = njobs // nw
    pairs = jobs_per_w // 2
    mesh = plsc.VectorSubcoreMesh(core_axis_name="c", subcore_axis_name="s")

    @functools.partial(
        pl.kernel,
        mesh=mesh,
        out_type=jax.ShapeDtypeStruct((e, 128), F32),
        scratch_types=[
            pltpu.VMEM((JOB,), jnp.int32),
            pltpu.VMEM((JOB,), jnp.int32),
            pltpu.VMEM((JOB,), jnp.int32),
            pltpu.VMEM((JOB,), jnp.int32),
            pltpu.VMEM((JOB, 128), F32),
            pltpu.VMEM((JOB, 128), F32),
            pltpu.VMEM((JOB, 128), F32),
            pltpu.VMEM((JOB, 128), F32),
            pltpu.SemaphoreType.DMA,
            pltpu.SemaphoreType.DMA,
            pltpu.SemaphoreType.DMA,
            pltpu.SemaphoreType.DMA,
        ],
        compiler_params=pltpu.CompilerParams(needs_layout_passes=False),
    )
    def gather_k(t1_hbm, t2_hbm, ii_hbm, jj_hbm, g_hbm,
                 iia_v, iib_v, jja_v, jjb_v, g1a_v, g1b_v, g2a_v, g2b_v,
                 s1a, s1b, s2a, s2b):
        cid = lax.axis_index("c")
        sid = lax.axis_index("s")
        wid = sid * nc + cid
        wbase = wid * jobs_per_w * JOB

        def add_rows(g1_v, g2_v):
            def add_row(r, carry2):
                for c in range(8):
                    sl = pl.ds(c * 16, 16)
                    g1_v[r, sl] = g1_v[r, sl] + g2_v[r, sl]
                return carry2

            lax.fori_loop(0, JOB, add_row, 0)

        def pair(t, carry):
            base = wbase + t * (2 * JOB)
            pltpu.sync_copy(ii_hbm.at[pl.ds(base, JOB)], iia_v)
            pltpu.sync_copy(jj_hbm.at[pl.ds(base, JOB)], jja_v)
            pltpu.sync_copy(ii_hbm.at[pl.ds(base + JOB, JOB)], iib_v)
            pltpu.sync_copy(jj_hbm.at[pl.ds(base + JOB, JOB)], jjb_v)
            d1a = pltpu.async_copy(t1_hbm.at[iia_v], g1a_v, s1a)
            d2a = pltpu.async_copy(t2_hbm.at[jja_v], g2a_v, s2a)
            d1b = pltpu.async_copy(t1_hbm.at[iib_v], g1b_v, s1b)
            d2b = pltpu.async_copy(t2_hbm.at[jjb_v], g2b_v, s2b)

            d1a.wait()
            d2a.wait()
            add_rows(g1a_v, g2a_v)
            pltpu.sync_copy(g1a_v, g_hbm.at[pl.ds(base, JOB)])
            d1b.wait()
            d2b.wait()
            add_rows(g1b_v, g2b_v)
            pltpu.sync_copy(g1b_v, g_hbm.at[pl.ds(base + JOB, JOB)])
            return carry

        lax.fori_loop(0, pairs, pair, 0)

    return gather_k


def _edge_body(g_ref, d2_ref, w1r_ref, w2_ref, b2_ref, m_ref):
    dist = jnp.sqrt(d2_ref[...])
    centers = (lax.broadcasted_iota(jnp.int32, (1, 32), 1).astype(F32)
               * (5.0 / 31.0))
    rbf = jnp.exp(-10.0 * (dist - centers) ** 2)
    t = g_ref[...] + jnp.dot(rbf, w1r_ref[...], preferred_element_type=F32)
    m = jnp.dot(jax.nn.relu(t), w2_ref[...], preferred_element_type=F32)
    m_ref[...] = m + b2_ref[...]


def _k_edge(g, d2, w1r, w2, b2, e):
    bn = 2560
    return pl.pallas_call(
        _edge_body,
        grid=(e // bn,),
        in_specs=[
            pl.BlockSpec((bn, 128), lambda i: (i, 0)),
            pl.BlockSpec((bn, 1), lambda i: (i, 0)),
            pl.BlockSpec((32, 128), lambda i: (0, 0)),
            pl.BlockSpec((128, 128), lambda i: (0, 0)),
            pl.BlockSpec((128,), lambda i: (0,)),
        ],
        out_specs=pl.BlockSpec((bn, 128), lambda i: (i, 0)),
        out_shape=jax.ShapeDtypeStruct((e, 128), F32),
    )(g, d2, w1r, w2, b2)


def _make_scatter(e, n, nc, ns):
    nw = nc * ns
    pj = 400
    jobs_per_sub = e // pj // ns
    half = n // nc
    trash = half
    zsub = 5
    zspan = half // zsub
    zrows = 200
    mesh = plsc.VectorSubcoreMesh(core_axis_name="c", subcore_axis_name="s")

    @functools.partial(
        pl.kernel,
        mesh=mesh,
        out_type=[
            jax.ShapeDtypeStruct((n, 128), F32),
            jax.ShapeDtypeStruct((nw * n,), F32),
        ],
        scratch_types=[
            pltpu.VMEM((pj,), jnp.int32),
            pltpu.VMEM((pj, 128), F32),
            pltpu.VMEM((n,), F32),
            pltpu.VMEM_SHARED((half + 8, 128), F32),
        ],
        compiler_params=pltpu.CompilerParams(needs_layout_passes=False),
    )
    def scatter_k(m_hbm, ii_hbm, out_hbm, deg_hbm,
                  ii0_v, m0_v, deg_v, acc_sh):
        cid = lax.axis_index("c")
        sid = lax.axis_index("s")
        wid = sid * nc + cid
        lo = cid * half
        sbase = sid * jobs_per_sub * pj

        zvec = jnp.zeros((16,), dtype=F32)

        def zrow(r, carry):
            for c in range(8):
                m0_v[r, pl.ds(c * 16, 16)] = zvec
            return carry

        lax.fori_loop(0, zrows, zrow, 0)

        def zdeg(r, carry):
            deg_v[pl.ds(r * 16, 16)] = zvec
            return carry

        lax.fori_loop(0, n // 16, zdeg, 0)

        @pl.when(sid < zsub)
        def _zero():
            def zcopy(k, carry):
                pltpu.sync_copy(
                    m0_v.at[pl.ds(0, zrows)],
                    acc_sh.at[pl.ds(sid * zspan + k * zrows, zrows)])
                return carry

            lax.fori_loop(0, zspan // zrows, zcopy, 0)

        plsc.subcore_barrier()

        def remap(ii_v):
            for s in range(pj // 16):
                sl = pl.ds(s * 16, 16)
                i16 = ii_v[sl]
                loc = i16 - lo
                own = (loc >= 0) & (loc < half)
                plsc.addupdate_scatter(
                    deg_v, [i16], jnp.where(own, 1.0, 0.0).astype(F32))
                ii_v[sl] = jnp.where(own, loc, trash)

        def body(u, carry):
            b0 = sbase + u * pj
            pltpu.sync_copy(ii_hbm.at[pl.ds(b0, pj)], ii0_v)
            pltpu.sync_copy(m_hbm.at[pl.ds(b0, pj)], m0_v)
            remap(ii0_v)
            pltpu.sync_copy(m0_v, acc_sh.at[ii0_v], add=True)
            return carry

        lax.fori_loop(0, jobs_per_sub, body, 0)
        plsc.subcore_barrier()

        pltpu.sync_copy(deg_v, deg_hbm.at[pl.ds(wid * n, n)])

        @pl.when(sid < zsub)
        def _writeout():
            def wcopy(k, carry):
                r0 = sid * zspan + k * zrows
                pltpu.sync_copy(acc_sh.at[pl.ds(r0, zrows)],
                                m0_v.at[pl.ds(0, zrows)])
                pltpu.sync_copy(m0_v.at[pl.ds(0, zrows)],
                                out_hbm.at[pl.ds(lo + r0, zrows)])
                return carry

            lax.fori_loop(0, zspan // zrows, wcopy, 0)

    return scatter_k


def _update_body(h_ref, p_ref, r_ref, w3a_ref, w3b_ref, b3_ref, w4_ref,
                 b4_ref, out_ref):
    aggr = p_ref[...] * r_ref[...]
    t = (jnp.dot(h_ref[...], w3a_ref[...], preferred_element_type=F32)
         + jnp.dot(aggr, w3b_ref[...], preferred_element_type=F32)
         + b3_ref[...])
    out_ref[...] = (jnp.dot(jax.nn.relu(t), w4_ref[...],
                            preferred_element_type=F32) + b4_ref[...])


def _k_update(h, p, recip, w3a, w3b, b3, w4, b4, n):
    bn = 1000
    return pl.pallas_call(
        _update_body,
        grid=(n // bn,),
        in_specs=[
            pl.BlockSpec((bn, 128), lambda i: (i, 0)),
            pl.BlockSpec((bn, 128), lambda i: (i, 0)),
            pl.BlockSpec((bn, 1), lambda i: (i, 0)),
            pl.BlockSpec((128, 128), lambda i: (0, 0)),
            pl.BlockSpec((128, 128), lambda i: (0, 0)),
            pl.BlockSpec((128,), lambda i: (0,)),
            pl.BlockSpec((128, 128), lambda i: (0, 0)),
            pl.BlockSpec((128,), lambda i: (0,)),
        ],
        out_specs=pl.BlockSpec((bn, 128), lambda i: (i, 0)),
        out_shape=jax.ShapeDtypeStruct((n, 128), F32),
    )(h, p, recip, w3a, w3b, b3, w4, b4)


def kernel(h, x, edge_index, W1, b1, W2, b2, W3, b3, W4, b4):
    n = h.shape[0]
    e = edge_index.shape[1]
    try:
        info = plsc.get_sparse_core_info()
        nc, ns = info.num_cores, info.num_subcores
    except Exception:
        nc, ns = 2, 16

    ii = edge_index[0].astype(jnp.int32)
    jj = edge_index[1].astype(jnp.int32)
    xf = jnp.pad(x.astype(F32), ((0, 0), (0, 1))).reshape(-1)
    w1a = W1[:128]
    w1b = W1[128:256]
    w1r = W1[256:288]
    w3a = W3[:128]
    w3b = W3[128:256]

    t1, t2 = _k_pre(h, w1a, w1b, b1, n)
    d2 = _make_d2(e, n, nc, ns)(ii, jj, xf)
    g = _make_gather(e, n, nc, ns)(t1, t2, ii, jj)
    m = _k_edge(g, d2.reshape(e, 1), w1r, W2, b2, e)
    p, deg = _make_scatter(e, n, nc, ns)(m, ii)
    recip = (1.0 / jnp.clip(deg.reshape(nc * ns, n).sum(axis=0),
                            1.0, None))[:, None]
    h_new = _k_update(h, p, recip, w3a, w3b, b3, W4, b4, n)
    return (h_new, x)


# speedup vs baseline: 4.7938x; 1.0593x over previous
import functools

import jax
import jax.numpy as jnp
from jax import lax
from jax.experimental import pallas as pl
from jax.experimental.pallas import tpu as pltpu
from jax.experimental.pallas import tpu_sc as plsc

F32 = jnp.float32
JOB = 200


def _pre_body(h_ref, w1a_ref, w1b_ref, b1_ref, t1_ref, t2_ref):
    h = h_ref[...]
    t1_ref[...] = jnp.dot(h, w1a_ref[...], preferred_element_type=F32) \
        + b1_ref[...]
    t2_ref[...] = jnp.dot(h, w1b_ref[...], preferred_element_type=F32)


def _k_pre(h, w1a, w1b, b1, n):
    bn = 2000
    return pl.pallas_call(
        _pre_body,
        grid=(n // bn,),
        in_specs=[
            pl.BlockSpec((bn, 128), lambda i: (i, 0)),
            pl.BlockSpec((128, 128), lambda i: (0, 0)),
            pl.BlockSpec((128, 128), lambda i: (0, 0)),
            pl.BlockSpec((128,), lambda i: (0,)),
        ],
        out_specs=[
            pl.BlockSpec((bn, 128), lambda i: (i, 0)),
            pl.BlockSpec((bn, 128), lambda i: (i, 0)),
        ],
        out_shape=[
            jax.ShapeDtypeStruct((n, 128), F32),
            jax.ShapeDtypeStruct((n, 128), F32),
        ],
    )(h, w1a, w1b, b1)


def _make_d2(e, n, nc, ns):
    nw = nc * ns
    job2 = 2000
    jobs_per_w = e // job2 // nw
    mesh = plsc.VectorSubcoreMesh(core_axis_name="c", subcore_axis_name="s")

    @functools.partial(
        pl.kernel,
        mesh=mesh,
        out_type=jax.ShapeDtypeStruct((e,), F32),
        scratch_types=[
            pltpu.VMEM((job2,), jnp.int32),
            pltpu.VMEM((job2,), jnp.int32),
            pltpu.VMEM((job2,), F32),
            pltpu.VMEM((4 * n,), F32),
        ],
        compiler_params=pltpu.CompilerParams(needs_layout_passes=False),
    )
    def d2_k(ii_hbm, jj_hbm, xf_hbm, d2_hbm, ii_v, jj_v, d2_v, xf_v):
        cid = lax.axis_index("c")
        sid = lax.axis_index("s")
        wid = sid * nc + cid
        pltpu.sync_copy(xf_hbm, xf_v)

        def job(k, carry):
            base = (wid * jobs_per_w + k) * job2
            pltpu.sync_copy(ii_hbm.at[pl.ds(base, job2)], ii_v)
            pltpu.sync_copy(jj_hbm.at[pl.ds(base, job2)], jj_v)

            def step(s, carry2):
                sl = pl.ds(s * 16, 16)
                i16 = ii_v[sl] * 4
                j16 = jj_v[sl] * 4
                acc = jnp.zeros((16,), dtype=F32)
                for c in range(3):
                    xi = plsc.load_gather(xf_v, [i16 + c])
                    xj = plsc.load_gather(xf_v, [j16 + c])
                    d = xi - xj
                    acc = acc + d * d
                d2_v[sl] = acc
                return carry2

            lax.fori_loop(0, job2 // 16, step, 0)
            pltpu.sync_copy(d2_v, d2_hbm.at[pl.ds(base, job2)])
            return carry

        lax.fori_loop(0, jobs_per_w, job, 0)

    return d2_k


def _make_gather(e, n, nc, ns):
    nw = nc * ns
    njobs = e // JOB
    jobs_per_w = njobs // nw
    pairs = jobs_per_w // 2
    mesh = plsc.VectorSubcoreMesh(core_axis_name="c", subcore_axis_name="s")

    @functools.partial(
        pl.kernel,
        mesh=mesh,
        out_type=jax.ShapeDtypeStruct((e, 128), F32),
        scratch_types=[
            pltpu.VMEM((10000,), jnp.int32),
            pltpu.VMEM((10000,), jnp.int32),
            pltpu.VMEM((JOB, 128), F32),
            pltpu.VMEM((JOB, 128), F32),
            pltpu.VMEM((JOB, 128), F32),
            pltpu.VMEM((JOB, 128), F32),
            pltpu.SemaphoreType.DMA,
            pltpu.SemaphoreType.DMA,
            pltpu.SemaphoreType.DMA,
            pltpu.SemaphoreType.DMA,
        ],
        compiler_params=pltpu.CompilerParams(needs_layout_passes=False),
    )
    def gather_k(t1_hbm, t2_hbm, ii_hbm, jj_hbm, g_hbm,
                 iiw_v, jjw_v, g1a_v, g1b_v, g2a_v, g2b_v,
                 s1a, s1b, s2a, s2b):
        cid = lax.axis_index("c")
        sid = lax.axis_index("s")
        wid = sid * nc + cid
        wbase = wid * jobs_per_w * JOB
        epw = jobs_per_w * JOB
        pltpu.sync_copy(ii_hbm.at[pl.ds(wbase, epw)], iiw_v)
        pltpu.sync_copy(jj_hbm.at[pl.ds(wbase, epw)], jjw_v)

        def add_rows(g1_v, g2_v):
            def add_row(r, carry2):
                for rr in range(4):
                    for c in range(8):
                        sl = pl.ds(c * 16, 16)
                        g1_v[4 * r + rr, sl] = \
                            g1_v[4 * r + rr, sl] + g2_v[4 * r + rr, sl]
                return carry2

            lax.fori_loop(0, JOB // 4, add_row, 0)

        def pair(t, carry):
            base = wbase + t * (2 * JOB)
            off = t * (2 * JOB)
            d1a = pltpu.async_copy(
                t1_hbm.at[iiw_v.at[pl.ds(off, JOB)]], g1a_v, s1a)
            d2a = pltpu.async_copy(
                t2_hbm.at[jjw_v.at[pl.ds(off, JOB)]], g2a_v, s2a)
            d1b = pltpu.async_copy(
                t1_hbm.at[iiw_v.at[pl.ds(off + JOB, JOB)]], g1b_v, s1b)
            d2b = pltpu.async_copy(
                t2_hbm.at[jjw_v.at[pl.ds(off + JOB, JOB)]], g2b_v, s2b)

            d1a.wait()
            d2a.wait()
            add_rows(g1a_v, g2a_v)
            pltpu.sync_copy(g1a_v, g_hbm.at[pl.ds(base, JOB)])
            d1b.wait()
            d2b.wait()
            add_rows(g1b_v, g2b_v)
            pltpu.sync_copy(g1b_v, g_hbm.at[pl.ds(base + JOB, JOB)])
            return carry

        lax.fori_loop(0, pairs, pair, 0)

    return gather_k


def _edge_body(g_ref, d2_ref, w1r_ref, w2_ref, b2_ref, m_ref):
    dist = jnp.sqrt(d2_ref[...])
    centers = (lax.broadcasted_iota(jnp.int32, (1, 32), 1).astype(F32)
               * (5.0 / 31.0))
    rbf = jnp.exp(-10.0 * (dist - centers) ** 2)
    t = g_ref[...] + jnp.dot(rbf, w1r_ref[...], preferred_element_type=F32)
    m = jnp.dot(jax.nn.relu(t), w2_ref[...], preferred_element_type=F32)
    m_ref[...] = m + b2_ref[...]


def _k_edge(g, d2, w1r, w2, b2, e):
    bn = 2560
    return pl.pallas_call(
        _edge_body,
        grid=(e // bn,),
        in_specs=[
            pl.BlockSpec((bn, 128), lambda i: (i, 0)),
            pl.BlockSpec((bn, 1), lambda i: (i, 0)),
            pl.BlockSpec((32, 128), lambda i: (0, 0)),
            pl.BlockSpec((128, 128), lambda i: (0, 0)),
            pl.BlockSpec((128,), lambda i: (0,)),
        ],
        out_specs=pl.BlockSpec((bn, 128), lambda i: (i, 0)),
        out_shape=jax.ShapeDtypeStruct((e, 128), F32),
    )(g, d2, w1r, w2, b2)


def _make_scatter(e, n, nc, ns):
    nw = nc * ns
    pj = 400
    jobs_per_sub = e // pj // ns
    half = n // nc
    trash = half
    zsub = 5
    zspan = half // zsub
    zrows = 200
    mesh = plsc.VectorSubcoreMesh(core_axis_name="c", subcore_axis_name="s")

    @functools.partial(
        pl.kernel,
        mesh=mesh,
        out_type=[
            jax.ShapeDtypeStruct((n, 128), F32),
            jax.ShapeDtypeStruct((nw * n,), F32),
        ],
        scratch_types=[
            pltpu.VMEM((pj,), jnp.int32),
            pltpu.VMEM((pj, 128), F32),
            pltpu.VMEM((n,), F32),
            pltpu.VMEM_SHARED((half + 8, 128), F32),
        ],
        compiler_params=pltpu.CompilerParams(needs_layout_passes=False),
    )
    def scatter_k(m_hbm, ii_hbm, out_hbm, deg_hbm,
                  ii0_v, m0_v, deg_v, acc_sh):
        cid = lax.axis_index("c")
        sid = lax.axis_index("s")
        wid = sid * nc + cid
        lo = cid * half
        sbase = sid * jobs_per_sub * pj

        zvec = jnp.zeros((16,), dtype=F32)

        def zrow(r, carry):
            for c in range(8):
                m0_v[r, pl.ds(c * 16, 16)] = zvec
            return carry

        lax.fori_loop(0, zrows, zrow, 0)

        def zdeg(r, carry):
            deg_v[pl.ds(r * 16, 16)] = zvec
            return carry

        lax.fori_loop(0, n // 16, zdeg, 0)

        @pl.when(sid < zsub)
        def _zero():
            def zcopy(k, carry):
                pltpu.sync_copy(
                    m0_v.at[pl.ds(0, zrows)],
                    acc_sh.at[pl.ds(sid * zspan + k * zrows, zrows)])
                return carry

            lax.fori_loop(0, zspan // zrows, zcopy, 0)

        plsc.subcore_barrier()

        def remap(ii_v):
            for s in range(pj // 16):
                sl = pl.ds(s * 16, 16)
                i16 = ii_v[sl]
                loc = i16 - lo
                own = (loc >= 0) & (loc < half)
                plsc.addupdate_scatter(
                    deg_v, [i16], jnp.where(own, 1.0, 0.0).astype(F32))
                ii_v[sl] = jnp.where(own, loc, trash)

        def body(u, carry):
            b0 = sbase + u * pj
            pltpu.sync_copy(ii_hbm.at[pl.ds(b0, pj)], ii0_v)
            pltpu.sync_copy(m_hbm.at[pl.ds(b0, pj)], m0_v)
            remap(ii0_v)
            pltpu.sync_copy(m0_v, acc_sh.at[ii0_v], add=True)
            return carry

        lax.fori_loop(0, jobs_per_sub, body, 0)
        plsc.subcore_barrier()

        pltpu.sync_copy(deg_v, deg_hbm.at[pl.ds(wid * n, n)])

        @pl.when(sid < zsub)
        def _writeout():
            def wcopy(k, carry):
                r0 = sid * zspan + k * zrows
                pltpu.sync_copy(acc_sh.at[pl.ds(r0, zrows)],
                                m0_v.at[pl.ds(0, zrows)])
                pltpu.sync_copy(m0_v.at[pl.ds(0, zrows)],
                                out_hbm.at[pl.ds(lo + r0, zrows)])
                return carry

            lax.fori_loop(0, zspan // zrows, wcopy, 0)

    return scatter_k


def _update_body(h_ref, p_ref, r_ref, w3a_ref, w3b_ref, b3_ref, w4_ref,
                 b4_ref, out_ref):
    aggr = p_ref[...] * r_ref[...]
    t = (jnp.dot(h_ref[...], w3a_ref[...], preferred_element_type=F32)
         + jnp.dot(aggr, w3b_ref[...], preferred_element_type=F32)
         + b3_ref[...])
    out_ref[...] = (jnp.dot(jax.nn.relu(t), w4_ref[...],
                            preferred_element_type=F32) + b4_ref[...])


def _k_update(h, p, recip, w3a, w3b, b3, w4, b4, n):
    bn = 1000
    return pl.pallas_call(
        _update_body,
        grid=(n // bn,),
        in_specs=[
            pl.BlockSpec((bn, 128), lambda i: (i, 0)),
            pl.BlockSpec((bn, 128), lambda i: (i, 0)),
            pl.BlockSpec((bn, 1), lambda i: (i, 0)),
            pl.BlockSpec((128, 128), lambda i: (0, 0)),
            pl.BlockSpec((128, 128), lambda i: (0, 0)),
            pl.BlockSpec((128,), lambda i: (0,)),
            pl.BlockSpec((128, 128), lambda i: (0, 0)),
            pl.BlockSpec((128,), lambda i: (0,)),
        ],
        out_specs=pl.BlockSpec((bn, 128), lambda i: (i, 0)),
        out_shape=jax.ShapeDtypeStruct((n, 128), F32),
    )(h, p, recip, w3a, w3b, b3, w4, b4)


def kernel(h, x, edge_index, W1, b1, W2, b2, W3, b3, W4, b4):
    n = h.shape[0]
    e = edge_index.shape[1]
    try:
        info = plsc.get_sparse_core_info()
        nc, ns = info.num_cores, info.num_subcores
    except Exception:
        nc, ns = 2, 16

    ii = edge_index[0].astype(jnp.int32)
    jj = edge_index[1].astype(jnp.int32)
    xf = jnp.pad(x.astype(F32), ((0, 0), (0, 1))).reshape(-1)
    w1a = W1[:128]
    w1b = W1[128:256]
    w1r = W1[256:288]
    w3a = W3[:128]
    w3b = W3[128:256]

    t1, t2 = _k_pre(h, w1a, w1b, b1, n)
    d2 = _make_d2(e, n, nc, ns)(ii, jj, xf)
    g = _make_gather(e, n, nc, ns)(t1, t2, ii, jj)
    m = _k_edge(g, d2.reshape(e, 1), w1r, W2, b2, e)
    p, deg = _make_scatter(e, n, nc, ns)(m, ii)
    recip = (1.0 / jnp.clip(deg.reshape(nc * ns, n).sum(axis=0),
                            1.0, None))[:, None]
    h_new = _k_update(h, p, recip, w3a, w3b, b3, W4, b4, n)
    return (h_new, x)
